# baseline (speedup 1.0000x reference)
import jax
import jax.numpy as jnp
from jax.experimental import pallas as pl

_CHUNK = 2000
_ZBLK = 6000


def _body(zc_ref, table_ref, out1_ref, out2_ref):
    i = pl.program_id(0)
    out2_ref[...] = jnp.zeros_like(out2_ref)

    @pl.when(i < 5)
    def _():
        idx = zc_ref[...]
        tv = table_ref[...]
        v = tv.shape[0]
        onehot = (idx == jax.lax.broadcasted_iota(
            jnp.int32, (idx.shape[0], v), 1))
        out1_ref[...] = jax.lax.dot_general(
            onehot.astype(jnp.float32), tv,
            dimension_numbers=(((1,), (0,)), ((), ())),
            preferred_element_type=jnp.float32)


def kernel(z, graph, edges_dist, orientation, table):
    del orientation
    zi = z.astype(jnp.int32)
    B = zi.shape[0]
    V, D = table.shape
    E = graph.shape[0]
    zc = zi.reshape(B, 1)
    n_blk = (E * 3) // _ZBLK
    node_scalars, flat_zeros = pl.pallas_call(
        _body,
        grid=(n_blk,),
        in_specs=[
            pl.BlockSpec((_CHUNK, 1), lambda i: (jnp.minimum(i, 4), 0)),
            pl.BlockSpec((V, D), lambda i: (0, 0)),
        ],
        out_specs=[
            pl.BlockSpec((_CHUNK, D), lambda i: (jnp.minimum(i, 4), 0)),
            pl.BlockSpec((_ZBLK, D), lambda i: (i, 0)),
        ],
        out_shape=[
            jax.ShapeDtypeStruct((B, D), jnp.float32),
            jax.ShapeDtypeStruct((E * 3, D), edges_dist.dtype),
        ],
    )(zc, table)
    node_vectors = flat_zeros.reshape(E, 3, D)
    return (node_scalars, node_vectors)


# speedup vs baseline: 7.5513x; 7.5513x over previous
import functools

import jax
import jax.numpy as jnp
from jax.experimental import pallas as pl

_CHUNK = 2000


def _gather_body(zc_ref, table_ref, out_ref):
    idx = zc_ref[...]
    tv = table_ref[...]
    v = tv.shape[0]
    onehot = (idx == jax.lax.broadcasted_iota(jnp.int32, (idx.shape[0], v), 1))
    out_ref[...] = jax.lax.dot_general(
        onehot.astype(jnp.float32), tv,
        dimension_numbers=(((1,), (0,)), ((), ())),
        preferred_element_type=jnp.float32)


def _tc_gather(table, idx):
    B = idx.shape[0]
    V, D = table.shape
    zc = idx.reshape(B, 1)
    grid = (B // _CHUNK,)
    return pl.pallas_call(
        _gather_body,
        grid=grid,
        in_specs=[
            pl.BlockSpec((_CHUNK, 1), lambda i: (i, 0)),
            pl.BlockSpec((V, D), lambda i: (0, 0)),
        ],
        out_specs=pl.BlockSpec((_CHUNK, D), lambda i: (i, 0)),
        out_shape=jax.ShapeDtypeStruct((B, D), jnp.float32),
    )(zc, table)


def kernel(z, graph, edges_dist, orientation, table):
    del orientation
    zi = z.astype(jnp.int32)
    node_scalars = _tc_gather(table, zi)
    node_vectors = jnp.zeros((graph.shape[0], 3, table.shape[1]),
                             dtype=edges_dist.dtype)
    return (node_scalars, node_vectors)


# speedup vs baseline: 7.8534x; 1.0400x over previous
import jax
import jax.numpy as jnp
from jax.experimental import pallas as pl

_CHUNK = 1280


def _gather_body(zrow_ref, table_ref, out_ref):
    zrow = zrow_ref[...]
    tv = table_ref[...]
    v = tv.shape[0]
    onehot_t = (zrow == jax.lax.broadcasted_iota(
        jnp.int32, (v, zrow.shape[1]), 0)).astype(jnp.bfloat16)
    t_hi = tv.astype(jnp.bfloat16)
    t_lo = (tv - t_hi.astype(jnp.float32)).astype(jnp.bfloat16)
    dims = (((0,), (0,)), ((), ()))
    out_ref[...] = (
        jax.lax.dot_general(onehot_t, t_hi, dimension_numbers=dims,
                            preferred_element_type=jnp.float32)
        + jax.lax.dot_general(onehot_t, t_lo, dimension_numbers=dims,
                              preferred_element_type=jnp.float32))


def _tc_gather(table, idx):
    B = idx.shape[0]
    V, D = table.shape
    zr = idx.reshape(1, B)
    grid = ((B + _CHUNK - 1) // _CHUNK,)
    return pl.pallas_call(
        _gather_body,
        grid=grid,
        in_specs=[
            pl.BlockSpec((1, _CHUNK), lambda i: (0, i)),
            pl.BlockSpec((V, D), lambda i: (0, 0)),
        ],
        out_specs=pl.BlockSpec((_CHUNK, D), lambda i: (i, 0)),
        out_shape=jax.ShapeDtypeStruct((B, D), jnp.float32),
    )(zr, table)


def kernel(z, graph, edges_dist, orientation, table):
    del orientation
    zi = z.astype(jnp.int32)
    node_scalars = _tc_gather(table, zi)
    node_vectors = jnp.zeros((graph.shape[0], 3, table.shape[1]),
                             dtype=edges_dist.dtype)
    return (node_scalars, node_vectors)


# speedup vs baseline: 8.0081x; 1.0197x over previous
import jax
import jax.numpy as jnp
from jax.experimental import pallas as pl
from jax.experimental.pallas import tpu as pltpu

_CHUNK = 2048


def _gather_body(zrow_ref, table_ref, out_ref):
    zrow = zrow_ref[...].reshape(1, -1)
    tv = table_ref[...]
    v = tv.shape[0]
    onehot_t = (zrow == jax.lax.broadcasted_iota(
        jnp.int32, (v, zrow.shape[1]), 0)).astype(jnp.bfloat16)
    t_hi = tv.astype(jnp.bfloat16)
    t_lo = (tv - t_hi.astype(jnp.float32)).astype(jnp.bfloat16)
    dims = (((0,), (0,)), ((), ()))
    out_ref[...] = (
        jax.lax.dot_general(onehot_t, t_hi, dimension_numbers=dims,
                            preferred_element_type=jnp.float32)
        + jax.lax.dot_general(onehot_t, t_lo, dimension_numbers=dims,
                              preferred_element_type=jnp.float32))


def _tc_gather(table, idx):
    B = idx.shape[0]
    V, D = table.shape
    grid = ((B + _CHUNK - 1) // _CHUNK,)
    return pl.pallas_call(
        _gather_body,
        grid=grid,
        in_specs=[
            pl.BlockSpec((_CHUNK,), lambda i: (i,)),
            pl.BlockSpec((V, D), lambda i: (0, 0)),
        ],
        compiler_params=pltpu.CompilerParams(
            fuse_transposed_lhs_in_matmul=True),
        out_specs=pl.BlockSpec((_CHUNK, D), lambda i: (i, 0)),
        out_shape=jax.ShapeDtypeStruct((B, D), jnp.float32),
    )(idx, table)


def kernel(z, graph, edges_dist, orientation, table):
    del orientation
    zi = z.astype(jnp.int32)
    node_scalars = _tc_gather(table, zi)
    node_vectors = jnp.zeros((graph.shape[0], 3, table.shape[1]),
                             dtype=edges_dist.dtype)
    return (node_scalars, node_vectors)
